# Initial kernel scaffold; baseline (speedup 1.0000x reference)
import jax
import jax.numpy as jnp
from jax.experimental import pallas as pl


def kernel(h, edge_index, e, emb, W0, a0, gamma0, beta0, W1, a1, gamma1, beta1, W2, a2, gamma2, beta2, W3, a3, gamma3, beta3, M0, c0, M1, c1, M2, c2):
    raise NotImplementedError("write your pallas kernel here")


# speedup vs baseline: 1.2024x; 1.2024x over previous
import jax
import jax.numpy as jnp
from jax.experimental import pallas as pl

_N_BLK = 2000
_E_BLK = 8000


def _embed_body(idx_ref, emb_ref, out_ref):
    idx = idx_ref[...]
    ids = jax.lax.broadcasted_iota(jnp.int32, (idx.shape[0], emb_ref.shape[0]), 1)
    onehot = (idx == ids).astype(jnp.float32)
    out_ref[...] = jnp.dot(onehot, emb_ref[...], preferred_element_type=jnp.float32)


def _embed(h_idx, emb):
    n = h_idx.shape[0]
    na, hd = emb.shape
    grid = n // _N_BLK
    return pl.pallas_call(
        _embed_body,
        grid=(grid,),
        in_specs=[
            pl.BlockSpec((_N_BLK, 1), lambda i: (i, 0)),
            pl.BlockSpec((na, hd), lambda i: (0, 0)),
        ],
        out_specs=pl.BlockSpec((_N_BLK, hd), lambda i: (i, 0)),
        out_shape=jax.ShapeDtypeStruct((n, hd), jnp.float32),
    )(h_idx.astype(jnp.int32).reshape(n, 1), emb)


def _node_body(h_ref, w_ref, al_ref, ar_ref, wh_ref, el_ref, er_ref):
    h = h_ref[...]
    wh = jnp.dot(h, w_ref[...], preferred_element_type=jnp.float32)
    wh_ref[...] = wh
    el_ref[...] = jnp.dot(wh, al_ref[...], preferred_element_type=jnp.float32)
    er_ref[...] = jnp.dot(wh, ar_ref[...], preferred_element_type=jnp.float32)


def _node_transform(h, W, a):
    n, hd = h.shape
    k = W.shape[1]
    nh = a.shape[0]
    od = a.shape[1] // 2
    eye = jnp.eye(nh, dtype=jnp.float32)
    Al = (a[:, :od, None] * eye[:, None, :]).reshape(nh * od, nh)
    Ar = (a[:, od:, None] * eye[:, None, :]).reshape(nh * od, nh)
    grid = n // _N_BLK
    return pl.pallas_call(
        _node_body,
        grid=(grid,),
        in_specs=[
            pl.BlockSpec((_N_BLK, hd), lambda i: (i, 0)),
            pl.BlockSpec((hd, k), lambda i: (0, 0)),
            pl.BlockSpec((k, nh), lambda i: (0, 0)),
            pl.BlockSpec((k, nh), lambda i: (0, 0)),
        ],
        out_specs=[
            pl.BlockSpec((_N_BLK, k), lambda i: (i, 0)),
            pl.BlockSpec((_N_BLK, nh), lambda i: (i, 0)),
            pl.BlockSpec((_N_BLK, nh), lambda i: (i, 0)),
        ],
        out_shape=[
            jax.ShapeDtypeStruct((n, k), jnp.float32),
            jax.ShapeDtypeStruct((n, nh), jnp.float32),
            jax.ShapeDtypeStruct((n, nh), jnp.float32),
        ],
    )(h, W, Al, Ar)


def _colsum_body(x_ref, s_ref, ss_ref):
    i = pl.program_id(0)

    @pl.when(i == 0)
    def _():
        s_ref[...] = jnp.zeros_like(s_ref)
        ss_ref[...] = jnp.zeros_like(ss_ref)

    x = x_ref[...]
    s_ref[...] += jnp.sum(x, axis=0, keepdims=True)
    ss_ref[...] += jnp.sum(x * x, axis=0, keepdims=True)


def _colsums(x, blk):
    n, k = x.shape
    grid = n // blk
    return pl.pallas_call(
        _colsum_body,
        grid=(grid,),
        in_specs=[pl.BlockSpec((blk, k), lambda i: (i, 0))],
        out_specs=[
            pl.BlockSpec((1, k), lambda i: (0, 0)),
            pl.BlockSpec((1, k), lambda i: (0, 0)),
        ],
        out_shape=[
            jax.ShapeDtypeStruct((1, k), jnp.float32),
            jax.ShapeDtypeStruct((1, k), jnp.float32),
        ],
    )(x)


def _norm_body(x_ref, h_ref, mu_ref, inv_ref, g_ref, b_ref, o_ref):
    y = (x_ref[...] - mu_ref[...]) * inv_ref[...] * g_ref[...] + b_ref[...]
    y = jnp.where(y > 0, y, jnp.exp(jnp.minimum(y, 0.0)) - 1.0)
    o_ref[...] = y + h_ref[...]


def _norm_elu_res(x, h, mu, inv, gamma, beta):
    n, k = x.shape
    grid = n // _N_BLK
    return pl.pallas_call(
        _norm_body,
        grid=(grid,),
        in_specs=[
            pl.BlockSpec((_N_BLK, k), lambda i: (i, 0)),
            pl.BlockSpec((_N_BLK, k), lambda i: (i, 0)),
            pl.BlockSpec((1, k), lambda i: (0, 0)),
            pl.BlockSpec((1, k), lambda i: (0, 0)),
            pl.BlockSpec((1, k), lambda i: (0, 0)),
            pl.BlockSpec((1, k), lambda i: (0, 0)),
        ],
        out_specs=pl.BlockSpec((_N_BLK, k), lambda i: (i, 0)),
        out_shape=jax.ShapeDtypeStruct((n, k), jnp.float32),
    )(x, h, mu, inv, gamma.reshape(1, k), beta.reshape(1, k))


def _mlp_body(x_ref, m0_ref, c0_ref, m1_ref, c1_ref, m2_ref, c2_ref, o_ref):
    x = x_ref[...]
    x = jnp.maximum(jnp.dot(x, m0_ref[...], preferred_element_type=jnp.float32) + c0_ref[...], 0.0)
    x = jnp.maximum(jnp.dot(x, m1_ref[...], preferred_element_type=jnp.float32) + c1_ref[...], 0.0)
    o_ref[...] = jnp.dot(x, m2_ref[...], preferred_element_type=jnp.float32) + c2_ref[...]


def _mlp(x, M0, c0, M1, c1, M2, c2):
    return pl.pallas_call(
        _mlp_body,
        out_shape=jax.ShapeDtypeStruct((1, M2.shape[1]), jnp.float32),
    )(x, M0, c0.reshape(1, -1), M1, c1.reshape(1, -1), M2, c2.reshape(1, -1))


def _gat_layer(h, src, dst, W, a, gamma, beta):
    n = h.shape[0]
    nh = a.shape[0]
    od = a.shape[1] // 2
    Wh, el, er = _node_transform(h, W, a)
    sc = jax.nn.leaky_relu(el[src] + er[dst], 0.2)
    smax = jax.ops.segment_max(sc, dst, num_segments=n)
    smax = jnp.where(jnp.isfinite(smax), smax, 0.0)
    ex = jnp.exp(sc - smax[dst])
    den = jax.ops.segment_sum(ex, dst, num_segments=n)
    alpha = ex / (den[dst] + 1e-9)
    msg = Wh[src].reshape(-1, nh, od) * alpha[:, :, None]
    out = jax.ops.segment_sum(msg, dst, num_segments=n).reshape(n, nh * od)
    s, ss = _colsums(out, _N_BLK)
    mu = s / n
    var = ss / n - mu * mu
    inv = 1.0 / jnp.sqrt(var + 1e-5)
    out = _norm_elu_res(out, h, mu, inv, gamma, beta)
    return out, alpha


def kernel(h, edge_index, e, emb, W0, a0, gamma0, beta0, W1, a1, gamma1, beta1, W2, a2, gamma2, beta2, W3, a3, gamma3, beta3, M0, c0, M1, c1, M2, c2):
    src, dst = edge_index[0], edge_index[1]
    x = _embed(h, emb)
    layers = [(W0, a0, gamma0, beta0), (W1, a1, gamma1, beta1),
              (W2, a2, gamma2, beta2), (W3, a3, gamma3, beta3)]
    top_feat = None
    for i, (W, a, g, b) in enumerate(layers):
        x, alpha = _gat_layer(x, src, dst, W, a, g, b)
        if i == 2:
            asum, _ = _colsums(alpha, _E_BLK)
            top_feat = asum
    n = x.shape[0]
    hs, _ = _colsums(x, _N_BLK)
    hg = jnp.concatenate([hs / n, top_feat], axis=1)
    return _mlp(hg, M0, c0, M1, c1, M2, c2)
